# baseline (speedup 1.0000x reference)
import jax
import jax.numpy as jnp
from jax.experimental import pallas as pl
from jax.experimental.pallas import tpu as pltpu

_NCLS = 1000
_MAXP = 100
_FDIM = 128
_CB = 40
_STEPS = 13
_MCH = 13
_MB = 4


def _body(cid_ref, protos_ref, protos_hbm, protos_out, counts_out,
          rsems, *bufs):
    i = pl.program_id(0)

    def rd(j):
        return pltpu.make_async_copy(
            protos_hbm.at[pl.ds(480 + j * _CB, _CB)], bufs[j % _MB],
            rsems.at[j % _MB])

    counts_out[...] = jnp.broadcast_to(
        protos_ref[0, :, 0].reshape(1, _MAXP).astype(jnp.int32), counts_out.shape)

    for j in range(_MCH):
        @pl.when(i == j)
        def _():
            rd(j).start()

    for j in range(_MCH):
        @pl.when(i == min(j + 2, _STEPS - 1))
        def _():
            rd(j).wait()


def kernel(features, prototypes, counts, class_id):
    cid = jnp.atleast_1d(jnp.asarray(class_id, jnp.int32))
    grid_spec = pltpu.PrefetchScalarGridSpec(
        num_scalar_prefetch=1,
        grid=(_STEPS,),
        in_specs=[
            pl.BlockSpec((_CB, _MAXP, _FDIM), lambda i, s: (i, 0, 0)),
            pl.BlockSpec(memory_space=pltpu.MemorySpace.HBM),
        ],
        out_specs=[
            pl.BlockSpec(memory_space=pltpu.MemorySpace.HBM),
            pl.BlockSpec((_NCLS, _MAXP), lambda i, s: (0, 0)),
        ],
        scratch_shapes=[
            pltpu.SemaphoreType.DMA((_MB,)),
        ] + [pltpu.VMEM((_CB, _MAXP, _FDIM), jnp.float32)] * _MB,
    )
    return pl.pallas_call(
        _body,
        grid_spec=grid_spec,
        out_shape=(
            jax.ShapeDtypeStruct((_NCLS, _MAXP, _FDIM), jnp.float32),
            jax.ShapeDtypeStruct((_NCLS, _MAXP), jnp.int32),
        ),
        compiler_params=pltpu.CompilerParams(
            dimension_semantics=("arbitrary",),
        ),
    )(cid, prototypes, prototypes)


# speedup vs baseline: 1.2264x; 1.2264x over previous
import jax
import jax.numpy as jnp
from jax import lax
from jax.experimental import pallas as pl
from jax.experimental.pallas import tpu as pltpu

_NCLS = 1000
_MAXP = 100
_FDIM = 128


def _body(cid_ref, feat_hbm, protos_in, counts_in, protos_out, counts_out,
          featv, normv, onesv, sem_f, sem_row, sem_cnt):
    cid = cid_ref[0]

    feat_in = pltpu.make_async_copy(feat_hbm.at[pl.ds(0, 104)], featv, sem_f)
    feat_in.start()
    onesv[...] = jnp.ones((8, _MAXP), jnp.int32)
    feat_in.wait()

    f = featv[...]
    norm = jnp.sqrt(jnp.sum(f * f, axis=1, keepdims=True))
    normv[...] = (f / jnp.maximum(norm, 1e-12))[:_MAXP]

    row_wr = pltpu.make_async_copy(normv, protos_out.at[cid], sem_row)
    cnt_wr = pltpu.make_async_copy(
        onesv.at[pl.ds(0, 1)], counts_out.at[pl.ds(cid, 1)], sem_cnt)
    row_wr.start()
    cnt_wr.start()
    row_wr.wait()
    cnt_wr.wait()


def kernel(features, prototypes, counts, class_id):
    cid = jnp.atleast_1d(jnp.asarray(class_id, jnp.int32))
    one_f = lax.optimization_barrier(jnp.float32(1.0))
    one_i = lax.optimization_barrier(jnp.int32(1))
    protos2 = prototypes * one_f
    counts2 = counts * one_i
    grid_spec = pltpu.PrefetchScalarGridSpec(
        num_scalar_prefetch=1,
        grid=(1,),
        in_specs=[
            pl.BlockSpec(memory_space=pltpu.MemorySpace.HBM),
            pl.BlockSpec(memory_space=pltpu.MemorySpace.HBM),
            pl.BlockSpec(memory_space=pltpu.MemorySpace.HBM),
        ],
        out_specs=[
            pl.BlockSpec(memory_space=pltpu.MemorySpace.HBM),
            pl.BlockSpec(memory_space=pltpu.MemorySpace.HBM),
        ],
        scratch_shapes=[
            pltpu.VMEM((104, _FDIM), jnp.float32),
            pltpu.VMEM((_MAXP, _FDIM), jnp.float32),
            pltpu.VMEM((8, _MAXP), jnp.int32),
            pltpu.SemaphoreType.DMA,
            pltpu.SemaphoreType.DMA,
            pltpu.SemaphoreType.DMA,
        ],
    )
    return pl.pallas_call(
        _body,
        grid_spec=grid_spec,
        out_shape=(
            jax.ShapeDtypeStruct((_NCLS, _MAXP, _FDIM), jnp.float32),
            jax.ShapeDtypeStruct((_NCLS, _MAXP), jnp.int32),
        ),
        input_output_aliases={2: 0, 3: 1},
        compiler_params=pltpu.CompilerParams(
            dimension_semantics=("arbitrary",),
        ),
    )(cid, features, protos2, counts2)
